# baseline (speedup 1.0000x reference)
import functools

import jax
import jax.numpy as jnp
from jax import lax
from jax.experimental import pallas as pl
from jax.experimental.pallas import tpu as pltpu
from jax.experimental.pallas import tpu_sc as plsc

VOCAB = 1000000
EMBED = 32
NUM_CORES = 2
NUM_SUBCORES = 16
NUM_WORKERS = NUM_CORES * NUM_SUBCORES
L = 16
BB = 128
TQ = 4


@functools.partial(jax.jit, static_argnums=(1, 2))
def _transpose_tokens(tok, n_pos, n_batch):
    mesh = plsc.VectorSubcoreMesh(core_axis_name="c", subcore_axis_name="s")

    @functools.partial(
        pl.kernel,
        mesh=mesh,
        out_type=jax.ShapeDtypeStruct((NUM_WORKERS, n_pos // 8, 8, BB),
                                      jnp.int32),
        scratch_types=[
            pltpu.VMEM((BB, n_pos), jnp.int32),
            pltpu.VMEM((n_pos // 8, 8, BB), jnp.int32),
        ],
        compiler_params=pltpu.CompilerParams(
            use_tc_tiling_on_sc=True, needs_layout_passes=False),
    )
    def k(tok_hbm, out_hbm, slab, tbuf):
        wid = lax.axis_index("s") * NUM_CORES + lax.axis_index("c")
        b0 = wid * BB
        iot = lax.iota(jnp.int32, L)
        bvecs = [iot + L * j for j in range(BB // L)]
        pltpu.sync_copy(tok_hbm.at[pl.ds(b0, BB), :], slab)

        def slab_body(i, c):
            vs = []
            for jj in range(2 * BB // L):
                tvec = jnp.full((L,), 0, jnp.int32) + (2 * i + jj // 8)
                vs.append(plsc.load_gather(slab, [bvecs[jj % 8], tvec]))
            for jj in range(2 * BB // L):
                t = 2 * i + jj // 8
                tbuf[t // 8, t % 8, pl.ds((jj % 8) * L, L)] = vs[jj]
            return c
        lax.fori_loop(0, n_pos // 2, slab_body, 0)
        pltpu.sync_copy(tbuf, out_hbm.at[wid])

    return k(tok)


@functools.partial(jax.jit, static_argnums=(2, 3))
def _gather_embed(idxt, table, n_pos, n_batch):
    mesh = plsc.VectorSubcoreMesh(core_axis_name="c", subcore_axis_name="s")
    n_units = n_pos // TQ
    assert n_units % 2 == 0 and n_batch // BB == NUM_WORKERS
    GB = TQ * BB

    @functools.partial(
        pl.kernel,
        mesh=mesh,
        out_type=jax.ShapeDtypeStruct((n_pos, EMBED // 8, n_batch * 8),
                                      jnp.float32),
        scratch_types=[
            pltpu.VMEM((n_pos // 8, 8, BB), jnp.int32),
            pltpu.VMEM((n_units, GB), jnp.int32),
        ] + [pltpu.VMEM((GB, EMBED), jnp.float32)] * 2
          + [pltpu.VMEM((TQ * EMBED * BB,), jnp.float32)] * 2
          + [pltpu.SemaphoreType.DMA] * 4,
        compiler_params=pltpu.CompilerParams(
            use_tc_tiling_on_sc=False, needs_layout_passes=False),
    )
    def k(idxt_hbm, table_hbm, out_hbm, idx3, idxs, r0, r1, t0, t1,
          sg0, sg1, so0, so1):
        rows = (r0, r1)
        tbuf = (t0, t1)
        sg = (sg0, sg1)
        so = (so0, so1)
        wid = lax.axis_index("s") * NUM_CORES + lax.axis_index("c")
        b0 = wid * BB

        iot = lax.iota(jnp.int32, L)
        pltpu.sync_copy(idxt_hbm.at[wid], idx3)

        def pack_body(u, c):
            for j in range(GB // L):
                t = u * TQ + j // 8
                idxs[u, pl.ds(j * L, L)] = \
                    idx3[t // 8, t % 8, pl.ds((j % 8) * L, L)]
            return c
        lax.fori_loop(0, n_units, pack_body, 0)

        def gather_start(u, rb):
            pltpu.async_copy(table_hbm.at[idxs.at[u]], rows[rb], sg[rb])

        def gather_wait(u, rb):
            pltpu.make_async_copy(table_hbm.at[idxs.at[u]],
                                  rows[rb], sg[rb]).wait()

        def wb_start(u, rb):
            for tl in range(TQ):
                for e8 in range(EMBED // 8):
                    pltpu.async_copy(
                        tbuf[rb].at[pl.ds((tl * 4 + e8) * 8 * BB, 8 * BB)],
                        out_hbm.at[u * TQ + tl, e8, pl.ds(b0 * 8, 8 * BB)],
                        so[rb])

        def wb_wait(u, rb):
            for tl in range(TQ):
                for e8 in range(EMBED // 8):
                    pltpu.make_async_copy(
                        tbuf[rb].at[pl.ds((tl * 4 + e8) * 8 * BB, 8 * BB)],
                        out_hbm.at[u * TQ + tl, e8, pl.ds(b0 * 8, 8 * BB)],
                        so[rb]).wait()

        perms = [(iot + d) & 15 for d in range(L)]
        fdst = [(perms[d] // 8) * 1024 + (perms[d] % 8) * BB + iot
                for d in range(L)]

        def transpose(rb):
            tb = tbuf[rb]
            def jb_body(jb, c):
                bv = iot + jb * L
                for eh in range(2):
                    sb_dst = ((jb // 8) * (4 * 8 * BB) + (eh * 2) * (8 * BB)
                              + (jb % 8) * L)
                    vs = []
                    for d in range(L):
                        vs.append(plsc.load_gather(
                            rows[rb], [bv, perms[d] + eh * L]))
                    for d in range(L):
                        plsc.store_scatter(tb, [fdst[d] + sb_dst], vs[d])
                return c
            lax.fori_loop(0, GB // L, jb_body, 0)

        gather_start(0, 0)

        def body(kk, carry):
            for rb in range(2):
                i = 2 * kk + rb
                if rb == 0:
                    gather_start(i + 1, 1)
                else:
                    @pl.when(kk < n_units // 2 - 1)
                    def _():
                        gather_start(i + 1, 0)
                gather_wait(i, rb)
                @pl.when(kk > 0)
                def _():
                    wb_wait(i - 2, rb)
                transpose(rb)
                wb_start(i, rb)
            return carry

        lax.fori_loop(0, n_units // 2, body, 0)
        wb_wait(n_units - 2, 0)
        wb_wait(n_units - 1, 1)

    return k(idxt, table)


def kernel(tokens, embedding_weight):
    n_batch, n_pos = tokens.shape
    idxt = _transpose_tokens(tokens, n_pos, n_batch)
    out3 = _gather_embed(idxt, embedding_weight, n_pos, n_batch)
    out = (out3.reshape(n_pos, EMBED // 8, n_batch // 128, 8, 128)
           .transpose(2, 4, 0, 1, 3)
           .reshape(n_batch, n_pos, EMBED))
    return out


# speedup vs baseline: 1.8321x; 1.8321x over previous
import functools

import jax
import jax.numpy as jnp
from jax import lax
from jax.experimental import pallas as pl
from jax.experimental.pallas import tpu as pltpu
from jax.experimental.pallas import tpu_sc as plsc

VOCAB = 1000000
EMBED = 32
NUM_CORES = 2
NUM_SUBCORES = 16
NUM_WORKERS = NUM_CORES * NUM_SUBCORES
L = 16
BB = 128
TQ = 4


@functools.partial(jax.jit, static_argnums=(1, 2))
def _transpose_tokens(tok, n_pos, n_batch):
    mesh = plsc.VectorSubcoreMesh(core_axis_name="c", subcore_axis_name="s")

    @functools.partial(
        pl.kernel,
        mesh=mesh,
        out_type=jax.ShapeDtypeStruct((NUM_WORKERS, n_pos // 8, 8, BB),
                                      jnp.int32),
        scratch_types=[
            pltpu.VMEM((BB, n_pos), jnp.int32),
            pltpu.VMEM((n_pos // 8, 8, BB), jnp.int32),
        ],
        compiler_params=pltpu.CompilerParams(
            use_tc_tiling_on_sc=True, needs_layout_passes=False),
    )
    def k(tok_hbm, out_hbm, slab, tbuf):
        wid = lax.axis_index("s") * NUM_CORES + lax.axis_index("c")
        b0 = wid * BB
        iot = lax.iota(jnp.int32, L)
        bvecs = [iot + L * j for j in range(BB // L)]
        pltpu.sync_copy(tok_hbm.at[pl.ds(b0, BB), :], slab)

        def slab_body(i, c):
            vs = []
            for jj in range(2 * BB // L):
                tvec = jnp.full((L,), 0, jnp.int32) + (2 * i + jj // 8)
                vs.append(plsc.load_gather(slab, [bvecs[jj % 8], tvec]))
            for jj in range(2 * BB // L):
                t = 2 * i + jj // 8
                tbuf[t // 8, t % 8, pl.ds((jj % 8) * L, L)] = vs[jj]
            return c
        lax.fori_loop(0, n_pos // 2, slab_body, 0)
        pltpu.sync_copy(tbuf, out_hbm.at[wid])

    return k(tok)


@functools.partial(jax.jit, static_argnums=(1,))
def _detile_table(tab_t, vocab):
    mesh = plsc.VectorSubcoreMesh(core_axis_name="c", subcore_axis_name="s")
    n_full = vocab // BB
    tail = vocab - n_full * BB
    per_w = n_full // NUM_WORKERS + 1

    @functools.partial(
        pl.kernel,
        mesh=mesh,
        out_type=jax.ShapeDtypeStruct((vocab // 4, BB), jnp.float32),
        scratch_types=[pltpu.VMEM((EMBED, BB), jnp.float32)] * 2
          + [pltpu.VMEM((EMBED, BB), jnp.float32)] * 2
          + [pltpu.SemaphoreType.DMA] * 4,
        compiler_params=pltpu.CompilerParams(
            use_tc_tiling_on_sc=True, needs_layout_passes=False),
    )
    def k(tab_hbm, tail_hbm, out_hbm, a0, a1, b0_, b1_, si0, si1, so0, so1):
        av = (a0, a1)
        bv = (b0_, b1_)
        si = (si0, si1)
        so = (so0, so1)
        wid = lax.axis_index("s") * NUM_CORES + lax.axis_index("c")
        iot = lax.iota(jnp.int32, L)
        perms = [(iot + d) & 15 for d in range(L)]
        rowb = [[((iot * EMBED + eh * L + perms[d]) // BB)
                 for d in range(L)] for eh in range(2)]
        colb = [[((iot * EMBED + eh * L + perms[d]) % BB)
                 for d in range(L)] for eh in range(2)]

        def chunk_of(i):
            return wid + NUM_WORKERS * i

        def in_start(kc, rb):
            pltpu.async_copy(tab_hbm.at[:, pl.ds(kc * BB, BB)], av[rb],
                             si[rb])

        def in_wait(kc, rb):
            pltpu.make_async_copy(tab_hbm.at[:, pl.ds(kc * BB, BB)], av[rb],
                                  si[rb]).wait()

        def out_start(kc, rb):
            pltpu.async_copy(bv[rb], out_hbm.at[pl.ds(kc * EMBED, EMBED), :],
                             so[rb])

        def out_wait(kc, rb):
            pltpu.make_async_copy(bv[rb],
                                  out_hbm.at[pl.ds(kc * EMBED, EMBED), :],
                                  so[rb]).wait()

        def transpose(rb, n_g):
            def g_body(g, c):
                vl = iot + L * g
                for eh in range(2):
                    vs = []
                    for d in range(L):
                        vs.append(plsc.load_gather(
                            av[rb], [perms[d] + eh * L, vl]))
                    for d in range(L):
                        plsc.store_scatter(
                            bv[rb], [rowb[eh][d] + g * (L * EMBED // BB),
                                     colb[eh][d]], vs[d])
                return c
            lax.fori_loop(0, n_g, g_body, 0)

        @pl.when(chunk_of(0) < n_full)
        def _():
            in_start(chunk_of(0), 0)
        @pl.when(chunk_of(1) < n_full)
        def _():
            in_start(chunk_of(1), 1)

        def body(ii, carry):
            for rb in range(2):
                i = 2 * ii + rb
                kc = chunk_of(i)
                @pl.when(kc < n_full)
                def _():
                    in_wait(kc, rb)
                    @pl.when(i >= 2)
                    def _():
                        out_wait(chunk_of(i - 2), rb)
                    transpose(rb, BB // L)
                    out_start(kc, rb)
                    @pl.when(chunk_of(i + 2) < n_full)
                    def _():
                        in_start(chunk_of(i + 2), rb)
            return carry

        lax.fori_loop(0, per_w // 2 + 1, body, 0)
        for i in range(per_w - 3, per_w):
            @pl.when((chunk_of(i) < n_full) & (chunk_of(i + 2) >= n_full))
            def _():
                out_wait(chunk_of(i), i % 2)

        if tail:
            @pl.when(wid == 0)
            def _():
                pltpu.sync_copy(tail_hbm, av[0])
                transpose(0, tail // L)
                pltpu.sync_copy(
                    bv[0].at[pl.ds(0, tail * EMBED // BB)],
                    out_hbm.at[pl.ds(n_full * EMBED, tail * EMBED // BB), :])

    tail_arr = jnp.pad(tab_t[:, n_full * BB:], ((0, 0), (0, BB - tail)))
    return k(tab_t, tail_arr)


@functools.partial(jax.jit, static_argnums=(2, 3))
def _gather_embed(idxt, table, n_pos, n_batch):
    mesh = plsc.VectorSubcoreMesh(core_axis_name="c", subcore_axis_name="s")
    n_units = n_pos // TQ
    assert n_units % 2 == 0 and n_batch // BB == NUM_WORKERS
    GB = TQ * BB

    @functools.partial(
        pl.kernel,
        mesh=mesh,
        out_type=jax.ShapeDtypeStruct((n_pos, EMBED // 8, n_batch * 8),
                                      jnp.float32),
        scratch_types=[
            pltpu.VMEM((n_pos // 8, 8, BB), jnp.int32),
            pltpu.VMEM((n_units, GB), jnp.int32),
        ] + [pltpu.VMEM((GB, EMBED), jnp.float32)] * 2
          + [pltpu.VMEM((TQ * EMBED * BB,), jnp.float32)] * 2
          + [pltpu.SemaphoreType.DMA] * 4,
        compiler_params=pltpu.CompilerParams(
            use_tc_tiling_on_sc=False, needs_layout_passes=False),
    )
    def k(idxt_hbm, table_hbm, out_hbm, idx3, idxs, r0, r1, t0, t1,
          sg0, sg1, so0, so1):
        rows = (r0, r1)
        tbuf = (t0, t1)
        sg = (sg0, sg1)
        so = (so0, so1)
        wid = lax.axis_index("s") * NUM_CORES + lax.axis_index("c")
        b0 = wid * BB

        iot = lax.iota(jnp.int32, L)
        pltpu.sync_copy(idxt_hbm.at[wid], idx3)

        def pack_body(u, c):
            for j in range(GB // L):
                t = u * TQ + j // 8
                idxs[u, pl.ds(j * L, L)] = \
                    idx3[t // 8, t % 8, pl.ds((j % 8) * L, L)]
            return c
        lax.fori_loop(0, n_units, pack_body, 0)

        def gather_start(u, rb):
            pltpu.async_copy(table_hbm.at[idxs.at[u]], rows[rb], sg[rb])

        def gather_wait(u, rb):
            pltpu.make_async_copy(table_hbm.at[idxs.at[u]],
                                  rows[rb], sg[rb]).wait()

        def wb_start(u, rb):
            for tl in range(TQ):
                for e8 in range(EMBED // 8):
                    pltpu.async_copy(
                        tbuf[rb].at[pl.ds((tl * 4 + e8) * 8 * BB, 8 * BB)],
                        out_hbm.at[u * TQ + tl, e8, pl.ds(b0 * 8, 8 * BB)],
                        so[rb])

        def wb_wait(u, rb):
            for tl in range(TQ):
                for e8 in range(EMBED // 8):
                    pltpu.make_async_copy(
                        tbuf[rb].at[pl.ds((tl * 4 + e8) * 8 * BB, 8 * BB)],
                        out_hbm.at[u * TQ + tl, e8, pl.ds(b0 * 8, 8 * BB)],
                        so[rb]).wait()

        perms = [(iot + d) & 15 for d in range(L)]
        fdst = [(perms[d] // 8) * 1024 + (perms[d] % 8) * BB + iot
                for d in range(L)]

        def transpose(rb):
            tb = tbuf[rb]
            def jb_body(jb, c):
                bv = iot + jb * L
                for eh in range(2):
                    sb_dst = ((jb // 8) * (4 * 8 * BB) + (eh * 2) * (8 * BB)
                              + (jb % 8) * L)
                    vs = []
                    for d in range(L):
                        vs.append(plsc.load_gather(
                            rows[rb], [bv, perms[d] + eh * L]))
                    for d in range(L):
                        plsc.store_scatter(tb, [fdst[d] + sb_dst], vs[d])
                return c
            lax.fori_loop(0, GB // L, jb_body, 0)

        gather_start(0, 0)

        def body(kk, carry):
            for rb in range(2):
                i = 2 * kk + rb
                if rb == 0:
                    gather_start(i + 1, 1)
                else:
                    @pl.when(kk < n_units // 2 - 1)
                    def _():
                        gather_start(i + 1, 0)
                gather_wait(i, rb)
                @pl.when(kk > 0)
                def _():
                    wb_wait(i - 2, rb)
                transpose(rb)
                wb_start(i, rb)
            return carry

        lax.fori_loop(0, n_units // 2, body, 0)
        wb_wait(n_units - 2, 0)
        wb_wait(n_units - 1, 1)

    return k(idxt, table)


def kernel(tokens, embedding_weight):
    n_batch, n_pos = tokens.shape
    idxt = _transpose_tokens(tokens, n_pos, n_batch)
    tab_lin = _detile_table(embedding_weight.T, VOCAB).reshape(VOCAB, EMBED)
    out3 = _gather_embed(idxt, tab_lin, n_pos, n_batch)
    out = (out3.reshape(n_pos, EMBED // 8, n_batch // 128, 8, 128)
           .transpose(2, 4, 0, 1, 3)
           .reshape(n_batch, n_pos, EMBED))
    return out
